# Initial kernel scaffold; baseline (speedup 1.0000x reference)
import jax
import jax.numpy as jnp
from jax.experimental import pallas as pl


def kernel(x, edge_index, W1_l, W1_r, b1, W2_l, W2_r, b2, Wc, bc):
    raise NotImplementedError("write your pallas kernel here")


# speedup vs baseline: 10.9042x; 10.9042x over previous
import functools
import jax
import jax.numpy as jnp
from jax import lax
from jax.experimental import pallas as pl
from jax.experimental.pallas import tpu as pltpu
from jax.experimental.pallas import tpu_sc as plsc

N = 100000
E = 1600000
GROUPS = 12800
E_PAD = GROUPS * 128
CG = 8
GPW1 = 400
NCH1 = 50
GPT2 = 800
NCH2 = 100
ZROWS = 6256
NACC = 100096

BLK = 2000


def _sc_scratch():
    return [
        pltpu.VMEM_SHARED((NACC, 16), jnp.float32),
        pltpu.VMEM((CG, 128), jnp.int32),
        pltpu.VMEM((CG, 128), jnp.int32),
        pltpu.VMEM((CG, 128, 16), jnp.float32),
        pltpu.SemaphoreType.DMA,
        pltpu.SemaphoreType.DMA,
    ]


_mesh = plsc.VectorSubcoreMesh(core_axis_name="c", subcore_axis_name="s")


@functools.partial(
    pl.kernel,
    out_type=jax.ShapeDtypeStruct((2, NACC, 16), jnp.float32),
    mesh=_mesh,
    scratch_types=_sc_scratch(),
    compiler_params=pltpu.CompilerParams(use_tc_tiling_on_sc=False),
)
def _sage_agg1(xpad_hbm, src_hbm, dst_hbm, zeros_hbm, out_hbm,
               acc, idxs, idxd, rows, gsem, ssem):
    c = lax.axis_index("c")
    s = lax.axis_index("s")
    w = s * 2 + c
    pltpu.sync_copy(zeros_hbm, acc.at[pl.ds(s * ZROWS, ZROWS)])
    plsc.subcore_barrier()

    def chunk(k, carry):
        gbase = w * GPW1 + k * CG
        pltpu.sync_copy(src_hbm.at[pl.ds(gbase, CG)], idxs)
        pltpu.sync_copy(dst_hbm.at[pl.ds(gbase, CG)], idxd)
        descs = [pltpu.async_copy(xpad_hbm.at[idxs.at[g]], rows.at[g], gsem)
                 for g in range(CG)]
        for d in descs:
            d.wait()
        descs = [pltpu.async_copy(rows.at[g], acc.at[idxd.at[g]], ssem, add=True)
                 for g in range(CG)]
        for d in descs:
            d.wait()
        return carry

    lax.fori_loop(0, NCH1, chunk, 0)
    plsc.subcore_barrier()
    pltpu.sync_copy(acc.at[pl.ds(s * ZROWS, ZROWS)],
                    out_hbm.at[c, pl.ds(s * ZROWS, ZROWS)])


@functools.partial(
    pl.kernel,
    out_type=jax.ShapeDtypeStruct((2, NACC, 16), jnp.float32),
    mesh=_mesh,
    scratch_types=_sc_scratch(),
    compiler_params=pltpu.CompilerParams(use_tc_tiling_on_sc=False),
)
def _sage_agg2(p0_hbm, p1_hbm, src_hbm, dst_hbm, zeros_hbm, out_hbm,
               acc, idxs, idxd, rows, gsem, ssem):
    c = lax.axis_index("c")
    s = lax.axis_index("s")
    pltpu.sync_copy(zeros_hbm, acc.at[pl.ds(s * ZROWS, ZROWS)])
    plsc.subcore_barrier()

    def chunk(k, carry):
        gbase = s * GPT2 + k * CG
        pltpu.sync_copy(src_hbm.at[pl.ds(gbase, CG)], idxs)
        pltpu.sync_copy(dst_hbm.at[pl.ds(gbase, CG)], idxd)

        @pl.when(c == 0)
        def _():
            descs = [pltpu.async_copy(p0_hbm.at[idxs.at[g]], rows.at[g], gsem)
                     for g in range(CG)]
            for d in descs:
                d.wait()

        @pl.when(c == 1)
        def _():
            descs = [pltpu.async_copy(p1_hbm.at[idxs.at[g]], rows.at[g], gsem)
                     for g in range(CG)]
            for d in descs:
                d.wait()

        descs = [pltpu.async_copy(rows.at[g], acc.at[idxd.at[g]], ssem, add=True)
                 for g in range(CG)]
        for d in descs:
            d.wait()
        return carry

    lax.fori_loop(0, NCH2, chunk, 0)
    plsc.subcore_barrier()
    pltpu.sync_copy(acc.at[pl.ds(s * ZROWS, ZROWS)],
                    out_hbm.at[c, pl.ds(s * ZROWS, ZROWS)])


def _tc1_body(agg_ref, x_ref, w1l_ref, w1r_ref, b1_ref, w2l_ref, w2r_ref,
              p0_ref, p1_ref, hr_ref, rdeg_ref):
    aggs = agg_ref[0] + agg_ref[1]
    rd = 1.0 / jnp.clip(aggs[:, 11:12], 1.0, None)
    mean = aggs * rd
    h = jnp.maximum(
        jnp.dot(mean, w1l_ref[...], preferred_element_type=jnp.float32)
        + jnp.dot(x_ref[...], w1r_ref[...], preferred_element_type=jnp.float32)
        + b1_ref[...][None, :], 0.0)
    p = jnp.dot(h, w2l_ref[...], preferred_element_type=jnp.float32)
    p0_ref[...] = p[:, :16]
    p1_ref[...] = p[:, 16:]
    hr_ref[...] = jnp.dot(h, w2r_ref[...], preferred_element_type=jnp.float32)
    rdeg_ref[...] = rd


def _tc2_body(agg_ref, hr_ref, rdeg_ref, b2_ref, wc_ref, bc_ref,
              logits_ref, emb_ref):
    a = jnp.concatenate([agg_ref[0], agg_ref[1]], axis=1)
    emb = jnp.maximum(a * rdeg_ref[...] + hr_ref[...] + b2_ref[...][None, :], 0.0)
    emb_ref[...] = emb
    logits_ref[...] = (
        jnp.dot(emb, wc_ref[...], preferred_element_type=jnp.float32)
        + bc_ref[...][None, :])


def kernel(x, edge_index, W1_l, W1_r, b1, W2_l, W2_r, b2, Wc, bc):
    src = edge_index[0].astype(jnp.int32)
    dst = edge_index[1].astype(jnp.int32)
    pad_e = E_PAD - E
    src2d = jnp.concatenate(
        [src, jnp.zeros((pad_e,), jnp.int32)]).reshape(GROUPS, 128)
    dst2d = jnp.concatenate(
        [dst, jnp.full((pad_e,), N, jnp.int32)]).reshape(GROUPS, 128)
    xpad = jnp.concatenate(
        [x, jnp.ones((N, 1), jnp.float32), jnp.zeros((N, 4), jnp.float32)],
        axis=1)
    w1l_p = jnp.pad(W1_l, ((0, 5), (0, 0)))
    w1r_p = jnp.pad(W1_r, ((0, 5), (0, 0)))
    zinit = jnp.zeros((ZROWS, 16), jnp.float32)

    agg1 = _sage_agg1(xpad, src2d, dst2d, zinit)

    grid = N // BLK
    p0, p1, hr, rdeg = pl.pallas_call(
        _tc1_body,
        grid=(grid,),
        in_specs=[
            pl.BlockSpec((2, BLK, 16), lambda i: (0, i, 0)),
            pl.BlockSpec((BLK, 16), lambda i: (i, 0)),
            pl.BlockSpec((16, 64), lambda i: (0, 0)),
            pl.BlockSpec((16, 64), lambda i: (0, 0)),
            pl.BlockSpec((64,), lambda i: (0,)),
            pl.BlockSpec((64, 32), lambda i: (0, 0)),
            pl.BlockSpec((64, 32), lambda i: (0, 0)),
        ],
        out_specs=[
            pl.BlockSpec((BLK, 16), lambda i: (i, 0)),
            pl.BlockSpec((BLK, 16), lambda i: (i, 0)),
            pl.BlockSpec((BLK, 32), lambda i: (i, 0)),
            pl.BlockSpec((BLK, 1), lambda i: (i, 0)),
        ],
        out_shape=[
            jax.ShapeDtypeStruct((N, 16), jnp.float32),
            jax.ShapeDtypeStruct((N, 16), jnp.float32),
            jax.ShapeDtypeStruct((N, 32), jnp.float32),
            jax.ShapeDtypeStruct((N, 1), jnp.float32),
        ],
    )(agg1, xpad, w1l_p, w1r_p, b1, W2_l, W2_r)

    agg2 = _sage_agg2(p0, p1, src2d, dst2d, zinit)

    logits, emb = pl.pallas_call(
        _tc2_body,
        grid=(grid,),
        in_specs=[
            pl.BlockSpec((2, BLK, 16), lambda i: (0, i, 0)),
            pl.BlockSpec((BLK, 32), lambda i: (i, 0)),
            pl.BlockSpec((BLK, 1), lambda i: (i, 0)),
            pl.BlockSpec((32,), lambda i: (0,)),
            pl.BlockSpec((32, 3), lambda i: (0, 0)),
            pl.BlockSpec((3,), lambda i: (0,)),
        ],
        out_specs=[
            pl.BlockSpec((BLK, 3), lambda i: (i, 0)),
            pl.BlockSpec((BLK, 32), lambda i: (i, 0)),
        ],
        out_shape=[
            jax.ShapeDtypeStruct((N, 3), jnp.float32),
            jax.ShapeDtypeStruct((N, 32), jnp.float32),
        ],
    )(agg2, hr, rdeg, b2, Wc, bc)

    return (logits, emb)
